# Initial kernel scaffold; baseline (speedup 1.0000x reference)
import jax
import jax.numpy as jnp
from jax.experimental import pallas as pl


def kernel(x, frame_rate, codebooks):
    raise NotImplementedError("write your pallas kernel here")


# speedup vs baseline: 1.2030x; 1.2030x over previous
import functools
import math

import jax
import jax.numpy as jnp
from jax.experimental import pallas as pl
from jax.experimental.pallas import tpu as pltpu

N_Q = 8
BINS = 1024
DIM = 256
T_BLK = 1500


def _rvq_body(x_ref, cb_ref, cbt_ref, quant_out_ref, codes_ref, rvq_ref,
              loss_ref):
    r = x_ref[0]
    qacc = jnp.zeros_like(r)
    iota = jax.lax.broadcasted_iota(jnp.int32, (BINS, T_BLK), 0)
    for q in range(N_Q):
        cb = cb_ref[q]
        cnorm = jnp.sum(cb * cb, axis=1, keepdims=True)
        rnorm = jnp.sum(r * r, axis=0, keepdims=True)
        mm = jax.lax.dot(
            cb.astype(jnp.bfloat16), r.astype(jnp.bfloat16),
            preferred_element_type=jnp.float32)
        dist = (rnorm - 2.0 * mm) + cnorm
        m = jnp.min(dist, axis=0, keepdims=True)
        idx = jnp.min(jnp.where(dist == m, iota, BINS), axis=0)
        onehot = (iota == idx[None, :]).astype(jnp.float32)
        quant = jax.lax.dot(
            cbt_ref[q], onehot, precision=jax.lax.Precision.HIGHEST,
            preferred_element_type=jnp.float32)
        r = r - quant
        qacc = qacc + quant
        codes_ref[0, q, :] = idx
        rvq_ref[q, 0] = quant
        loss_ref[0, 0, q, :] = jnp.sum(r * r, axis=0)
    quant_out_ref[0] = qacc


def kernel(x, frame_rate, codebooks):
    b, dim, t = x.shape
    n_q, bins, _ = codebooks.shape
    n_t = t // T_BLK
    cbt = codebooks.transpose(0, 2, 1)

    grid = (b,)
    quantized, codes, rvq, loss = pl.pallas_call(
        _rvq_body,
        grid=grid,
        in_specs=[
            pl.BlockSpec((1, dim, T_BLK), lambda i: (i, 0, 0)),
            pl.BlockSpec((n_q, bins, dim), lambda i: (0, 0, 0)),
            pl.BlockSpec((n_q, dim, bins), lambda i: (0, 0, 0)),
        ],
        out_specs=[
            pl.BlockSpec((1, dim, T_BLK), lambda i: (i, 0, 0)),
            pl.BlockSpec((1, n_q, T_BLK), lambda i: (i, 0, 0)),
            pl.BlockSpec((n_q, 1, dim, T_BLK), lambda i: (0, i, 0, 0)),
            pl.BlockSpec((1, 1, n_q, T_BLK), lambda i: (i, 0, 0, 0)),
        ],
        out_shape=[
            jax.ShapeDtypeStruct((b, dim, t), jnp.float32),
            jax.ShapeDtypeStruct((b, n_q, t), jnp.int32),
            jax.ShapeDtypeStruct((n_q, b, dim, t), jnp.float32),
            jax.ShapeDtypeStruct((b, n_t, n_q, T_BLK), jnp.float32),
        ],
        compiler_params=pltpu.CompilerParams(
            vmem_limit_bytes=128 * 1024 * 1024),
    )(x, codebooks, cbt)

    penalty = jnp.sum(loss) / jnp.float32(n_q * b * t * dim)
    bw = jnp.asarray(n_q * math.log2(bins) * frame_rate / 1000.0,
                     dtype=x.dtype)
    return quantized, codes, bw, penalty, rvq, x


# speedup vs baseline: 1.8990x; 1.5785x over previous
import functools
import math

import jax
import jax.numpy as jnp
from jax.experimental import pallas as pl
from jax.experimental.pallas import tpu as pltpu

N_Q = 8
BINS = 1024
DIM = 256
T_BLK = 1500


def _rvq_body(x_ref, cb_ref, cbt3_ref, cnorm_ref, quant_out_ref, codes_ref,
              rvq_ref, loss_ref):
    r = x_ref[0]
    qacc = jnp.zeros_like(r)
    iota = jax.lax.broadcasted_iota(jnp.int32, (BINS, T_BLK), 0)
    for q in range(N_Q):
        cnorm = cnorm_ref[q]
        rnorm = jnp.sum(r * r, axis=0, keepdims=True)
        mm = jax.lax.dot(
            cb_ref[q], r.astype(jnp.bfloat16),
            preferred_element_type=jnp.float32)
        dist = (rnorm - 2.0 * mm) + cnorm
        m = jnp.min(dist, axis=0, keepdims=True)
        idx = jnp.min(jnp.where(dist == m, iota, BINS), axis=0)
        onehot = (iota == idx[None, :]).astype(jnp.bfloat16)
        threehot = jnp.concatenate([onehot, onehot, onehot], axis=0)
        quant = jax.lax.dot(cbt3_ref[q], threehot,
                            preferred_element_type=jnp.float32)
        r = r - quant
        qacc = qacc + quant
        codes_ref[0, q, :] = idx
        rvq_ref[q, 0] = quant
        loss_ref[0, 0, q, :] = jnp.sum(r * r, axis=0)
    quant_out_ref[0] = qacc


def kernel(x, frame_rate, codebooks):
    b, dim, t = x.shape
    n_q, bins, _ = codebooks.shape
    n_t = t // T_BLK
    cbt = codebooks.transpose(0, 2, 1)
    mask = jnp.uint32(0xFFFF0000)
    hi_f = jax.lax.bitcast_convert_type(
        jax.lax.bitcast_convert_type(cbt, jnp.uint32) & mask, jnp.float32)
    rem1 = cbt - hi_f
    mid_f = jax.lax.bitcast_convert_type(
        jax.lax.bitcast_convert_type(rem1, jnp.uint32) & mask, jnp.float32)
    lo_f = rem1 - mid_f
    cbt3 = jnp.concatenate([hi_f.astype(jnp.bfloat16),
                            mid_f.astype(jnp.bfloat16),
                            lo_f.astype(jnp.bfloat16)], axis=2)
    cnorm = jnp.sum(codebooks * codebooks, axis=-1)[:, :, None]
    cb_bf = codebooks.astype(jnp.bfloat16)

    grid = (b,)
    quantized, codes, rvq, loss = pl.pallas_call(
        _rvq_body,
        grid=grid,
        in_specs=[
            pl.BlockSpec((1, dim, T_BLK), lambda i: (i, 0, 0)),
            pl.BlockSpec((n_q, bins, dim), lambda i: (0, 0, 0)),
            pl.BlockSpec((n_q, dim, 3 * bins), lambda i: (0, 0, 0)),
            pl.BlockSpec((n_q, bins, 1), lambda i: (0, 0, 0)),
        ],
        out_specs=[
            pl.BlockSpec((1, dim, T_BLK), lambda i: (i, 0, 0)),
            pl.BlockSpec((1, n_q, T_BLK), lambda i: (i, 0, 0)),
            pl.BlockSpec((n_q, 1, dim, T_BLK), lambda i: (0, i, 0, 0)),
            pl.BlockSpec((1, 1, n_q, T_BLK), lambda i: (i, 0, 0, 0)),
        ],
        out_shape=[
            jax.ShapeDtypeStruct((b, dim, t), jnp.float32),
            jax.ShapeDtypeStruct((b, n_q, t), jnp.int32),
            jax.ShapeDtypeStruct((n_q, b, dim, t), jnp.float32),
            jax.ShapeDtypeStruct((b, n_t, n_q, T_BLK), jnp.float32),
        ],
        compiler_params=pltpu.CompilerParams(
            vmem_limit_bytes=128 * 1024 * 1024),
    )(x, cb_bf, cbt3, cnorm)

    penalty = jnp.sum(loss) / jnp.float32(n_q * b * t * dim)
    bw = jnp.asarray(n_q * math.log2(bins) * frame_rate / 1000.0,
                     dtype=x.dtype)
    return quantized, codes, bw, penalty, rvq, x


# speedup vs baseline: 1.9819x; 1.0437x over previous
import functools
import math

import jax
import jax.numpy as jnp
from jax.experimental import pallas as pl
from jax.experimental.pallas import tpu as pltpu

N_Q = 8
BINS = 1024
DIM = 256
T_BLK = 1500


T_SPLIT = 768


def _rvq_body(x_ref, cb_ref, cbt3_ref, cnorm_ref, quant_out_ref, codes_ref,
              rvq_ref, loss_ref):
    if 0 < T_SPLIT < T_BLK:
        chunks = [(0, T_SPLIT), (T_SPLIT, T_BLK - T_SPLIT)]
    else:
        chunks = [(0, T_BLK)]
    rs = [x_ref[0, :, lo:lo + n] for (lo, n) in chunks]
    qaccs = [jnp.zeros_like(rc) for rc in rs]
    iotas = [jax.lax.broadcasted_iota(jnp.int32, (BINS, n), 0)
             for (_, n) in chunks]
    for q in range(N_Q):
        cnorm = cnorm_ref[q]
        for ci, (lo, n) in enumerate(chunks):
            r = rs[ci]
            iota = iotas[ci]
            rnorm = jnp.sum(r * r, axis=0, keepdims=True)
            mm = jax.lax.dot(
                cb_ref[q], r.astype(jnp.bfloat16),
                preferred_element_type=jnp.float32)
            dist = (rnorm - 2.0 * mm) + cnorm
            m = jnp.min(dist, axis=0, keepdims=True)
            idx = jnp.min(jnp.where(dist == m, iota, BINS), axis=0)
            onehot = (iota == idx[None, :]).astype(jnp.bfloat16)
            threehot = jnp.concatenate([onehot, onehot, onehot], axis=0)
            quant = jax.lax.dot(cbt3_ref[q], threehot,
                                preferred_element_type=jnp.float32)
            r = r - quant
            rs[ci] = r
            qaccs[ci] = qaccs[ci] + quant
            codes_ref[0, q, lo:lo + n] = idx
            rvq_ref[q, 0, :, lo:lo + n] = quant
            loss_ref[0, 0, q, lo:lo + n] = jnp.sum(r * r, axis=0)
    for ci, (lo, n) in enumerate(chunks):
        quant_out_ref[0, :, lo:lo + n] = qaccs[ci]


def kernel(x, frame_rate, codebooks):
    b, dim, t = x.shape
    n_q, bins, _ = codebooks.shape
    n_t = t // T_BLK
    cbt = codebooks.transpose(0, 2, 1)
    mask = jnp.uint32(0xFFFF0000)
    hi_f = jax.lax.bitcast_convert_type(
        jax.lax.bitcast_convert_type(cbt, jnp.uint32) & mask, jnp.float32)
    rem1 = cbt - hi_f
    mid_f = jax.lax.bitcast_convert_type(
        jax.lax.bitcast_convert_type(rem1, jnp.uint32) & mask, jnp.float32)
    lo_f = rem1 - mid_f
    cbt3 = jnp.concatenate([hi_f.astype(jnp.bfloat16),
                            mid_f.astype(jnp.bfloat16),
                            lo_f.astype(jnp.bfloat16)], axis=2)
    cnorm = jnp.sum(codebooks * codebooks, axis=-1)[:, :, None]
    cb_bf = codebooks.astype(jnp.bfloat16)

    grid = (b,)
    quantized, codes, rvq, loss = pl.pallas_call(
        _rvq_body,
        grid=grid,
        in_specs=[
            pl.BlockSpec((1, dim, T_BLK), lambda i: (i, 0, 0)),
            pl.BlockSpec((n_q, bins, dim), lambda i: (0, 0, 0)),
            pl.BlockSpec((n_q, dim, 3 * bins), lambda i: (0, 0, 0)),
            pl.BlockSpec((n_q, bins, 1), lambda i: (0, 0, 0)),
        ],
        out_specs=[
            pl.BlockSpec((1, dim, T_BLK), lambda i: (i, 0, 0)),
            pl.BlockSpec((1, n_q, T_BLK), lambda i: (i, 0, 0)),
            pl.BlockSpec((n_q, 1, dim, T_BLK), lambda i: (0, i, 0, 0)),
            pl.BlockSpec((1, 1, n_q, T_BLK), lambda i: (i, 0, 0, 0)),
        ],
        out_shape=[
            jax.ShapeDtypeStruct((b, dim, t), jnp.float32),
            jax.ShapeDtypeStruct((b, n_q, t), jnp.int32),
            jax.ShapeDtypeStruct((n_q, b, dim, t), jnp.float32),
            jax.ShapeDtypeStruct((b, n_t, n_q, T_BLK), jnp.float32),
        ],
        compiler_params=pltpu.CompilerParams(
            vmem_limit_bytes=128 * 1024 * 1024),
    )(x, cb_bf, cbt3, cnorm)

    penalty = jnp.sum(loss) / jnp.float32(n_q * b * t * dim)
    bw = jnp.asarray(n_q * math.log2(bins) * frame_rate / 1000.0,
                     dtype=x.dtype)
    return quantized, codes, bw, penalty, rvq, x


# speedup vs baseline: 1.9946x; 1.0064x over previous
import functools
import math

import jax
import jax.numpy as jnp
from jax.experimental import pallas as pl
from jax.experimental.pallas import tpu as pltpu

N_Q = 8
BINS = 1024
DIM = 256
T_BLK = 1500


T_SPLIT = 768


def _rvq_body(x_ref, cb_ref, cbt3_ref, cnorm_ref, quant_out_ref, codes_ref,
              rvq_ref, loss_ref):
    if 0 < T_SPLIT < T_BLK:
        chunks = [(0, T_SPLIT), (T_SPLIT, T_BLK - T_SPLIT)]
    else:
        chunks = [(0, T_BLK)]
    rs = [x_ref[0, :, lo:lo + n] for (lo, n) in chunks]
    rnorms = [jnp.sum(rc * rc, axis=0, keepdims=True) for rc in rs]
    iotas = [jax.lax.broadcasted_iota(jnp.int32, (BINS, n), 0)
             for (_, n) in chunks]
    for q in range(N_Q):
        cnorm = cnorm_ref[q]
        for ci, (lo, n) in enumerate(chunks):
            r = rs[ci]
            iota = iotas[ci]
            rnorm = rnorms[ci]
            mm = jax.lax.dot(
                cb_ref[q], r.astype(jnp.bfloat16),
                preferred_element_type=jnp.float32)
            dist = (rnorm - 2.0 * mm) + cnorm
            m = jnp.min(dist, axis=0, keepdims=True)
            idx = jnp.min(jnp.where(dist == m, iota, BINS), axis=0)
            onehot = (iota == idx[None, :]).astype(jnp.bfloat16)
            threehot = jnp.concatenate([onehot, onehot, onehot], axis=0)
            quant = jax.lax.dot(cbt3_ref[q], threehot,
                                preferred_element_type=jnp.float32)
            r = r - quant
            rs[ci] = r
            rnorms[ci] = jnp.sum(r * r, axis=0, keepdims=True)
            codes_ref[0, q, lo:lo + n] = idx
            rvq_ref[q, 0, :, lo:lo + n] = quant
            loss_ref[0, 0, q, lo:lo + n] = rnorms[ci][0]
    for ci, (lo, n) in enumerate(chunks):
        quant_out_ref[0, :, lo:lo + n] = x_ref[0, :, lo:lo + n] - rs[ci]


def kernel(x, frame_rate, codebooks):
    b, dim, t = x.shape
    n_q, bins, _ = codebooks.shape
    n_t = t // T_BLK
    cbt = codebooks.transpose(0, 2, 1)
    mask = jnp.uint32(0xFFFF0000)
    hi_f = jax.lax.bitcast_convert_type(
        jax.lax.bitcast_convert_type(cbt, jnp.uint32) & mask, jnp.float32)
    rem1 = cbt - hi_f
    mid_f = jax.lax.bitcast_convert_type(
        jax.lax.bitcast_convert_type(rem1, jnp.uint32) & mask, jnp.float32)
    lo_f = rem1 - mid_f
    cbt3 = jnp.concatenate([hi_f.astype(jnp.bfloat16),
                            mid_f.astype(jnp.bfloat16),
                            lo_f.astype(jnp.bfloat16)], axis=2)
    cnorm = jnp.sum(codebooks * codebooks, axis=-1)[:, :, None]
    cb_bf = codebooks.astype(jnp.bfloat16)

    grid = (b,)
    quantized, codes, rvq, loss = pl.pallas_call(
        _rvq_body,
        grid=grid,
        in_specs=[
            pl.BlockSpec((1, dim, T_BLK), lambda i: (i, 0, 0)),
            pl.BlockSpec((n_q, bins, dim), lambda i: (0, 0, 0)),
            pl.BlockSpec((n_q, dim, 3 * bins), lambda i: (0, 0, 0)),
            pl.BlockSpec((n_q, bins, 1), lambda i: (0, 0, 0)),
        ],
        out_specs=[
            pl.BlockSpec((1, dim, T_BLK), lambda i: (i, 0, 0)),
            pl.BlockSpec((1, n_q, T_BLK), lambda i: (i, 0, 0)),
            pl.BlockSpec((n_q, 1, dim, T_BLK), lambda i: (0, i, 0, 0)),
            pl.BlockSpec((1, 1, n_q, T_BLK), lambda i: (i, 0, 0, 0)),
        ],
        out_shape=[
            jax.ShapeDtypeStruct((b, dim, t), jnp.float32),
            jax.ShapeDtypeStruct((b, n_q, t), jnp.int32),
            jax.ShapeDtypeStruct((n_q, b, dim, t), jnp.float32),
            jax.ShapeDtypeStruct((b, n_t, n_q, T_BLK), jnp.float32),
        ],
        compiler_params=pltpu.CompilerParams(
            vmem_limit_bytes=128 * 1024 * 1024),
    )(x, cb_bf, cbt3, cnorm)

    penalty = jnp.sum(loss) / jnp.float32(n_q * b * t * dim)
    bw = jnp.asarray(n_q * math.log2(bins) * frame_rate / 1000.0,
                     dtype=x.dtype)
    return quantized, codes, bw, penalty, rvq, x
